# baseline (speedup 1.0000x reference)
import functools

import jax
import jax.numpy as jnp
from jax import lax
from jax.experimental import pallas as pl
from jax.experimental.pallas import tpu as pltpu
from jax.experimental.pallas import tpu_sc as plsc

N_NODES = 10000
N_EDGES = 160000
D_FEAT = 256
CHANNELS = 128
N_LABELS = 40
NLP = 48

NC = 2
NS = 16
L = 16
NW = NC * NS

CH = 128
NCHEAVY = 58
NCLIGHT = 22
NCPH = 32
HEAVY0 = True
EPAD = (NCHEAVY + NCLIGHT) * NS * CH
NP = 10240
RPT = NP // NS


def _mm_body(x_ref, w_ref, o_ref):
    o_ref[...] = jnp.dot(x_ref[...], w_ref[...],
                         preferred_element_type=jnp.float32)


def _matmul(x, w, bm):
    m, k = x.shape
    n = w.shape[1]
    return pl.pallas_call(
        _mm_body,
        grid=(m // bm,),
        in_specs=[
            pl.BlockSpec((bm, k), lambda i: (i, 0)),
            pl.BlockSpec((k, n), lambda i: (0, 0)),
        ],
        out_specs=pl.BlockSpec((bm, n), lambda i: (i, 0)),
        out_shape=jax.ShapeDtypeStruct((m, n), jnp.float32),
    )(x, w)


def _relu_sum_body(p_ref, o_ref):
    o_ref[...] = jnp.maximum(p_ref[0] + p_ref[1], 0.0)


def _relu_sum(p, bm):
    _, m, k = p.shape
    return pl.pallas_call(
        _relu_sum_body,
        grid=(m // bm,),
        in_specs=[pl.BlockSpec((NC, bm, k), lambda i: (0, i, 0))],
        out_specs=pl.BlockSpec((bm, k), lambda i: (i, 0)),
        out_shape=jax.ShapeDtypeStruct((m, k), jnp.float32),
    )(p)


def _mm_softmax_body(p_ref, w_ref, o_ref):
    s = jnp.dot(p_ref[0] + p_ref[1], w_ref[...],
                preferred_element_type=jnp.float32)
    m = jnp.max(s, axis=1, keepdims=True)
    e = jnp.exp(s - m)
    o_ref[...] = e / jnp.sum(e, axis=1, keepdims=True)


def _mm_softmax(p, w, bm):
    _, m, k = p.shape
    n = w.shape[1]
    return pl.pallas_call(
        _mm_softmax_body,
        grid=(m // bm,),
        in_specs=[
            pl.BlockSpec((NC, bm, k), lambda i: (0, i, 0)),
            pl.BlockSpec((k, n), lambda i: (0, 0)),
        ],
        out_specs=pl.BlockSpec((bm, n), lambda i: (i, 0)),
        out_shape=jax.ShapeDtypeStruct((m, n), jnp.float32),
    )(p, w)


def _make_aggregate(d):
    mesh = plsc.VectorSubcoreMesh(core_axis_name="c", subcore_axis_name="s")

    @functools.partial(
        pl.kernel,
        mesh=mesh,
        out_type=jax.ShapeDtypeStruct((NC, NP, d), jnp.float32),
        scratch_types=[
            pltpu.VMEM((NCPH, CH), jnp.int32),
            pltpu.VMEM((NCPH, CH), jnp.int32),
            pltpu.VMEM((NCPH, CH), jnp.float32),
            pltpu.VMEM((CH, d), jnp.float32),
            pltpu.VMEM((CH, d), jnp.float32),
            pltpu.VMEM_SHARED((NP, d), jnp.float32),
            pltpu.SemaphoreType.DMA,
            pltpu.SemaphoreType.DMA,
        ],
    )
    def agg(h_hbm, src_hbm, dst_hbm, w_hbm, z_hbm, out_hbm,
            src_v, dst_v, w_v, rows_v0, rows_v1, acc, sem0, sem1):
        cid = lax.axis_index("c")
        sid = lax.axis_index("s")

        pltpu.sync_copy(z_hbm.at[pl.ds(sid * RPT, RPT)],
                        acc.at[pl.ds(sid * RPT, RPT)])
        plsc.subcore_barrier()

        my_heavy = (cid == 0) if HEAVY0 else (cid == 1)
        bufs = (rows_v0, rows_v1)
        sems = (sem0, sem1)

        def gather(i, b):
            return pltpu.make_async_copy(h_hbm.at[src_v.at[i]], bufs[b],
                                         sems[b])

        def process(i, b):
            gather(i, b).wait()

            def scale_group(g, carry2):
                wv = w_v[i, pl.ds(g * L, L)]
                rv = bufs[b]
                for k in range(L):
                    r = g * L + k
                    wk = wv[k]
                    for j in range(d // L):
                        sl = pl.ds(j * L, L)
                        rv[r, sl] = rv[r, sl] * wk
                return carry2

            lax.fori_loop(0, CH // L, scale_group, 0)
            pltpu.sync_copy(bufs[b], acc.at[dst_v.at[i]], add=True)

        def mk_pair(nch):
            def pair(j, carry):
                i0 = 2 * j
                gather(i0 + 1, 1).start()
                process(i0, 0)

                @pl.when(i0 + 2 < nch)
                def _():
                    gather(i0 + 2, 0).start()

                process(i0 + 1, 1)
                return carry
            return pair

        for p, (nh, nl) in enumerate(((NCPH, NCLIGHT), (NCHEAVY - NCPH, 0))):
            nch = jnp.where(my_heavy, nh, nl)
            npairs = nch >> 1
            pltpu.sync_copy(src_hbm.at[cid, p, sid], src_v)
            pltpu.sync_copy(dst_hbm.at[cid, p, sid], dst_v)
            pltpu.sync_copy(w_hbm.at[cid, p, sid], w_v)

            @pl.when(npairs > 0)
            def _():
                gather(0, 0).start()

            lax.fori_loop(0, npairs, mk_pair(nch), 0)

        plsc.subcore_barrier()
        pltpu.sync_copy(acc.at[pl.ds(sid * RPT, RPT)],
                        out_hbm.at[cid, pl.ds(sid * RPT, RPT)])

    return agg


_agg128 = _make_aggregate(CHANNELS)


@jax.jit
def _run(x, edge_index, edge_weight, W0, W1):
    src = edge_index[0].astype(jnp.int32)
    dst = edge_index[1].astype(jnp.int32)
    pad = EPAD - N_EDGES
    src = jnp.concatenate([src, jnp.zeros((pad,), jnp.int32)])
    dst = jnp.concatenate([dst, jnp.zeros((pad,), jnp.int32)])
    w = jnp.concatenate([edge_weight, jnp.zeros((pad,), jnp.float32)])

    nheavy = NS * NCHEAVY * CH

    def split(a):
        heavy = a[:nheavy].reshape(NS, NCHEAVY, CH)
        hp0 = heavy[:, :NCPH]
        hp1 = jnp.pad(heavy[:, NCPH:],
                      ((0, 0), (0, 2 * NCPH - NCHEAVY), (0, 0)))
        light = a[nheavy:].reshape(NS, NCLIGHT, CH)
        lp0 = jnp.pad(light, ((0, 0), (0, NCPH - NCLIGHT), (0, 0)))
        lp1 = jnp.zeros_like(lp0)
        heavy_ph = jnp.stack([hp0, hp1])
        light_ph = jnp.stack([lp0, lp1])
        if HEAVY0:
            return jnp.stack([heavy_ph, light_ph])
        return jnp.stack([light_ph, heavy_ph])

    src = split(src)
    dst = split(dst)
    w = split(w)

    z128 = jnp.zeros((NP, CHANNELS), jnp.float32)

    h0 = _matmul(x, W0, 1000)
    p0 = _agg128(h0, src, dst, w, z128)
    h1 = _relu_sum(p0, 1024)
    p1 = _agg128(h1, src, dst, w, z128)
    out = _mm_softmax(p1, W1, 1024)
    return out[:N_NODES]


def kernel(x, edge_index, edge_weight, W0, W1):
    return _run(x, edge_index, edge_weight, W0, W1)


# speedup vs baseline: 1.0087x; 1.0087x over previous
import functools

import jax
import jax.numpy as jnp
from jax import lax
from jax.experimental import pallas as pl
from jax.experimental.pallas import tpu as pltpu
from jax.experimental.pallas import tpu_sc as plsc

N_NODES = 10000
N_EDGES = 160000
D_FEAT = 256
CHANNELS = 128
N_LABELS = 40
NLP = 48

NC = 2
NS = 16
L = 16
NW = NC * NS

CH = 128
NCHEAVY = 60
NCLIGHT = 20
NCPH = 32
HEAVY0 = True
EPAD = (NCHEAVY + NCLIGHT) * NS * CH
NP = 10240
RPT = NP // NS


def _mm_body(x_ref, w_ref, o_ref):
    o_ref[...] = jnp.dot(x_ref[...], w_ref[...],
                         preferred_element_type=jnp.float32)


def _matmul(x, w, bm):
    m, k = x.shape
    n = w.shape[1]
    return pl.pallas_call(
        _mm_body,
        grid=(m // bm,),
        in_specs=[
            pl.BlockSpec((bm, k), lambda i: (i, 0)),
            pl.BlockSpec((k, n), lambda i: (0, 0)),
        ],
        out_specs=pl.BlockSpec((bm, n), lambda i: (i, 0)),
        out_shape=jax.ShapeDtypeStruct((m, n), jnp.float32),
    )(x, w)


def _relu_sum_body(p_ref, o_ref):
    o_ref[...] = jnp.maximum(p_ref[0] + p_ref[1], 0.0)


def _relu_sum(p, bm):
    _, m, k = p.shape
    return pl.pallas_call(
        _relu_sum_body,
        grid=(m // bm,),
        in_specs=[pl.BlockSpec((NC, bm, k), lambda i: (0, i, 0))],
        out_specs=pl.BlockSpec((bm, k), lambda i: (i, 0)),
        out_shape=jax.ShapeDtypeStruct((m, k), jnp.float32),
    )(p)


def _mm_softmax_body(p_ref, w_ref, o_ref):
    s = jnp.dot(p_ref[0] + p_ref[1], w_ref[...],
                preferred_element_type=jnp.float32)
    m = jnp.max(s, axis=1, keepdims=True)
    e = jnp.exp(s - m)
    o_ref[...] = e / jnp.sum(e, axis=1, keepdims=True)


def _mm_softmax(p, w, bm):
    _, m, k = p.shape
    n = w.shape[1]
    return pl.pallas_call(
        _mm_softmax_body,
        grid=(m // bm,),
        in_specs=[
            pl.BlockSpec((NC, bm, k), lambda i: (0, i, 0)),
            pl.BlockSpec((k, n), lambda i: (0, 0)),
        ],
        out_specs=pl.BlockSpec((bm, n), lambda i: (i, 0)),
        out_shape=jax.ShapeDtypeStruct((m, n), jnp.float32),
    )(p, w)


def _make_aggregate(d):
    mesh = plsc.VectorSubcoreMesh(core_axis_name="c", subcore_axis_name="s")

    @functools.partial(
        pl.kernel,
        mesh=mesh,
        out_type=jax.ShapeDtypeStruct((NC, NP, d), jnp.float32),
        scratch_types=[
            pltpu.VMEM((NCPH, CH), jnp.int32),
            pltpu.VMEM((NCPH, CH), jnp.int32),
            pltpu.VMEM((NCPH, CH), jnp.float32),
            pltpu.VMEM((CH, d), jnp.float32),
            pltpu.VMEM((CH, d), jnp.float32),
            pltpu.VMEM_SHARED((NP, d), jnp.float32),
            pltpu.SemaphoreType.DMA,
            pltpu.SemaphoreType.DMA,
        ],
    )
    def agg(h_hbm, src_hbm, dst_hbm, w_hbm, z_hbm, out_hbm,
            src_v, dst_v, w_v, rows_v0, rows_v1, acc, sem0, sem1):
        cid = lax.axis_index("c")
        sid = lax.axis_index("s")

        pltpu.sync_copy(z_hbm.at[pl.ds(sid * RPT, RPT)],
                        acc.at[pl.ds(sid * RPT, RPT)])
        plsc.subcore_barrier()

        my_heavy = (cid == 0) if HEAVY0 else (cid == 1)
        bufs = (rows_v0, rows_v1)
        sems = (sem0, sem1)

        def gather(i, b):
            return pltpu.make_async_copy(h_hbm.at[src_v.at[i]], bufs[b],
                                         sems[b])

        def process(i, b):
            gather(i, b).wait()

            def scale_group(g, carry2):
                wv = w_v[i, pl.ds(g * L, L)]
                rv = bufs[b]
                for k in range(L):
                    r = g * L + k
                    wk = wv[k]
                    for j in range(d // L):
                        sl = pl.ds(j * L, L)
                        rv[r, sl] = rv[r, sl] * wk
                return carry2

            lax.fori_loop(0, CH // L, scale_group, 0)
            pltpu.sync_copy(bufs[b], acc.at[dst_v.at[i]], add=True)

        def mk_pair(nch):
            def pair(j, carry):
                i0 = 2 * j
                gather(i0 + 1, 1).start()
                process(i0, 0)

                @pl.when(i0 + 2 < nch)
                def _():
                    gather(i0 + 2, 0).start()

                process(i0 + 1, 1)
                return carry
            return pair

        for p, (nh, nl) in enumerate(((NCPH, NCLIGHT), (NCHEAVY - NCPH, 0))):
            nch = jnp.where(my_heavy, nh, nl)
            npairs = nch >> 1
            pltpu.sync_copy(src_hbm.at[cid, p, sid], src_v)
            pltpu.sync_copy(dst_hbm.at[cid, p, sid], dst_v)
            pltpu.sync_copy(w_hbm.at[cid, p, sid], w_v)

            @pl.when(npairs > 0)
            def _():
                gather(0, 0).start()

            lax.fori_loop(0, npairs, mk_pair(nch), 0)

        plsc.subcore_barrier()
        pltpu.sync_copy(acc.at[pl.ds(sid * RPT, RPT)],
                        out_hbm.at[cid, pl.ds(sid * RPT, RPT)])

    return agg


_agg128 = _make_aggregate(CHANNELS)


@jax.jit
def _run(x, edge_index, edge_weight, W0, W1):
    src = edge_index[0].astype(jnp.int32)
    dst = edge_index[1].astype(jnp.int32)
    pad = EPAD - N_EDGES
    src = jnp.concatenate([src, jnp.zeros((pad,), jnp.int32)])
    dst = jnp.concatenate([dst, jnp.zeros((pad,), jnp.int32)])
    w = jnp.concatenate([edge_weight, jnp.zeros((pad,), jnp.float32)])

    nheavy = NS * NCHEAVY * CH

    def split(a):
        heavy = a[:nheavy].reshape(NS, NCHEAVY, CH)
        hp0 = heavy[:, :NCPH]
        hp1 = jnp.pad(heavy[:, NCPH:],
                      ((0, 0), (0, 2 * NCPH - NCHEAVY), (0, 0)))
        light = a[nheavy:].reshape(NS, NCLIGHT, CH)
        lp0 = jnp.pad(light, ((0, 0), (0, NCPH - NCLIGHT), (0, 0)))
        lp1 = jnp.zeros_like(lp0)
        heavy_ph = jnp.stack([hp0, hp1])
        light_ph = jnp.stack([lp0, lp1])
        if HEAVY0:
            return jnp.stack([heavy_ph, light_ph])
        return jnp.stack([light_ph, heavy_ph])

    src = split(src)
    dst = split(dst)
    w = split(w)

    z128 = jnp.zeros((NP, CHANNELS), jnp.float32)

    h0 = _matmul(x, W0, 1000)
    p0 = _agg128(h0, src, dst, w, z128)
    h1 = _relu_sum(p0, 1024)
    p1 = _agg128(h1, src, dst, w, z128)
    out = _mm_softmax(p1, W1, 1024)
    return out[:N_NODES]


def kernel(x, edge_index, edge_weight, W0, W1):
    return _run(x, edge_index, edge_weight, W0, W1)
